# baseline (speedup 1.0000x reference)
import jax
import jax.numpy as jnp
from jax import lax
from jax.experimental import pallas as pl

NUM_SERIES_C = 128
B_BLK = 256


def _norm_body(i_ref, loc_ref, scale_ref, x_ref, o_ref):
    iv = i_ref[0, 0, :]
    onehot = iv[:, None] == lax.broadcasted_iota(
        jnp.int32, (B_BLK, NUM_SERIES_C), 1
    )
    loc_row = loc_ref[0, :]
    scale_row = scale_ref[0, :]
    lg = jnp.sum(jnp.where(onehot, loc_row[None, :], 0.0), axis=1)
    sg = jnp.sum(jnp.where(onehot, scale_row[None, :], 0.0), axis=1)
    o_ref[...] = (x_ref[...] - lg[:, None]) / sg[:, None]


def kernel(x, i, loc, scale):
    bs, num_patch, out_len = x.shape
    row = num_patch * out_len
    nblk = bs // B_BLK
    x2 = x.reshape(bs, row)
    i3 = i.reshape(nblk, 1, B_BLK)
    loc2 = loc.reshape(1, -1)
    scale2 = scale.reshape(1, -1)

    out = pl.pallas_call(
        _norm_body,
        grid=(nblk,),
        in_specs=[
            pl.BlockSpec((1, 1, B_BLK), lambda b: (b, 0, 0)),
            pl.BlockSpec((1, NUM_SERIES_C), lambda b: (0, 0)),
            pl.BlockSpec((1, NUM_SERIES_C), lambda b: (0, 0)),
            pl.BlockSpec((B_BLK, row), lambda b: (b, 0)),
        ],
        out_specs=pl.BlockSpec((B_BLK, row), lambda b: (b, 0)),
        out_shape=jax.ShapeDtypeStruct((bs, row), x.dtype),
    )(i3, loc2, scale2, x2)
    return out.reshape(bs, num_patch, out_len)


# speedup vs baseline: 1.0003x; 1.0003x over previous
import jax
import jax.numpy as jnp
from jax import lax
from jax.experimental import pallas as pl

NUM_SERIES_C = 128
B_BLK = 256


def _norm_body(i_ref, loc_ref, scale_ref, x_ref, o_ref):
    iv = i_ref[0, 0, :]
    onehot = iv[:, None] == lax.broadcasted_iota(
        jnp.int32, (B_BLK, NUM_SERIES_C), 1
    )
    loc_row = loc_ref[0, :]
    scale_row = scale_ref[0, :]
    lg = jnp.sum(jnp.where(onehot, loc_row[None, :], 0.0), axis=1)
    sg = jnp.sum(jnp.where(onehot, scale_row[None, :], 0.0), axis=1)
    rg = 1.0 / sg
    o_ref[...] = (x_ref[...] - lg[:, None]) * rg[:, None]


def kernel(x, i, loc, scale):
    bs, num_patch, out_len = x.shape
    row = num_patch * out_len
    nblk = bs // B_BLK
    x2 = x.reshape(bs, row)
    i3 = i.reshape(nblk, 1, B_BLK)
    loc2 = loc.reshape(1, -1)
    scale2 = scale.reshape(1, -1)

    out = pl.pallas_call(
        _norm_body,
        grid=(nblk,),
        in_specs=[
            pl.BlockSpec((1, 1, B_BLK), lambda b: (b, 0, 0)),
            pl.BlockSpec((1, NUM_SERIES_C), lambda b: (0, 0)),
            pl.BlockSpec((1, NUM_SERIES_C), lambda b: (0, 0)),
            pl.BlockSpec((B_BLK, row), lambda b: (b, 0)),
        ],
        out_specs=pl.BlockSpec((B_BLK, row), lambda b: (b, 0)),
        out_shape=jax.ShapeDtypeStruct((bs, row), x.dtype),
    )(i3, loc2, scale2, x2)
    return out.reshape(bs, num_patch, out_len)
